# baseline (speedup 1.0000x reference)
import functools

import jax
import jax.numpy as jnp
from jax import lax
from jax.experimental import pallas as pl
from jax.experimental.pallas import tpu as pltpu
from jax.experimental.pallas import tpu_sc as plsc

_B = 16384
_D = 1000

_NC = 2
_NS = 16
_NW = _NC * _NS
_BPW = _B // _NW
_RCH = 32
_NCHK = _BPW // _RCH
_UR = 4
_NACC = 8
_NSL = 63
_TS = 992


@functools.partial(
    pl.kernel,
    mesh=plsc.VectorSubcoreMesh(core_axis_name="c", subcore_axis_name="s"),
    out_type=jax.ShapeDtypeStruct((_NW * 16,), jnp.float32),
    scratch_types=[
        pltpu.VMEM((2, _RCH, _D), jnp.float32),
        pltpu.VMEM((_D,), jnp.float32),
        pltpu.VMEM((_BPW,), jnp.int32),
        pltpu.VMEM((16,), jnp.float32),
        pltpu.SemaphoreType.DMA,
    ],
    compiler_params=pltpu.CompilerParams(use_tc_tiling_on_sc=True),
)
def _loss_kernel(x_hbm, tgt_hbm, w_hbm, out_hbm,
                 xb_v, w_v, tgt_v, o_v, ssem):
    iota16 = lax.broadcasted_iota(jnp.int32, (16,), 0)
    low8 = iota16 < 8
    wid = lax.axis_index("s") * _NC + lax.axis_index("c")
    row0 = wid * _BPW

    pltpu.sync_copy(tgt_hbm.at[pl.ds(row0, _BPW)], tgt_v)
    pltpu.sync_copy(w_hbm, w_v)

    ts_dyn = pl.multiple_of(jnp.int32(_TS) + wid * 0, 16)

    pltpu.async_copy(x_hbm.at[pl.ds(row0, _RCH), :], xb_v.at[0], ssem)

    def _chunk_body(k, carry):
        accs = list(carry[:_NACC])
        corr = carry[_NACC]
        half = lax.rem(k, 2)

        @pl.when(k + 1 < _NCHK)
        def _start_next():
            pltpu.async_copy(
                x_hbm.at[pl.ds(row0 + (k + 1) * _RCH, _RCH), :],
                xb_v.at[lax.rem(k + 1, 2)], ssem)

        pltpu.make_async_copy(
            x_hbm.at[pl.ds(0, _RCH), :], xb_v.at[0], ssem).wait()

        def _group_body(g, accs):
            accs = list(accs)
            n = 0
            for c in range(_NSL - 1):
                wv = w_v[pl.ds(c * 16, 16)]
                for u in range(_UR):
                    xv = xb_v[half, g * _UR + u, pl.ds(c * 16, 16)]
                    accs[n % _NACC] = accs[n % _NACC] + wv * (xv * xv)
                    n += 1
            wv = w_v[pl.ds(ts_dyn, 16)]
            for u in range(_UR):
                xv = xb_v[half, g * _UR + u, pl.ds(ts_dyn, 16)]
                accs[n % _NACC] = accs[n % _NACC] + jnp.where(
                    low8, wv * (xv * xv), 0.0)
                n += 1
            return tuple(accs)

        accs = lax.fori_loop(0, _RCH // _UR, _group_body, tuple(accs))

        for h16 in range(_RCH // 16):
            toff = pl.multiple_of(k * _RCH + h16 * 16, 16)
            t16 = tgt_v[pl.ds(toff, 16)]
            for u in range(16):
                t = t16[u]
                start = pl.multiple_of(t & ~jnp.int32(15), 16)
                xv = xb_v[half, h16 * 16 + u, pl.ds(start, 16)]
                wv = w_v[pl.ds(start, 16)]
                hit = iota16 == (t - start)
                corr = corr + jnp.where(hit, wv * (1.0 - 2.0 * xv), 0.0)
        return tuple(accs) + (corr,)

    init = tuple(jnp.zeros((16,), jnp.float32) for _ in range(_NACC + 1))
    carry = lax.fori_loop(0, _NCHK, _chunk_body, init)

    acc = carry[_NACC]
    for a in carry[:_NACC]:
        acc = acc + a
    o_v[...] = acc
    pltpu.sync_copy(o_v, out_hbm.at[pl.ds(wid * 16, 16)])


def kernel(inputs, targets, loss_weights):
    parts = _loss_kernel(inputs, targets, loss_weights)
    return jnp.sum(parts) / jnp.float32(_B * _D)


# speedup vs baseline: 1.3614x; 1.3614x over previous
import functools

import jax
import jax.numpy as jnp
from jax import lax
from jax.experimental import pallas as pl
from jax.experimental.pallas import tpu as pltpu
from jax.experimental.pallas import tpu_sc as plsc

_B = 16384
_D = 1000

_NC = 2
_NS = 16
_NW = _NC * _NS
_BPW = _B // _NW
_FPW = _BPW * _D
_SR = 2 * _D
_CH = 16 * _SR
_NCHK = _FPW // _CH
_NACC = 8
_NCHUNK = _BPW // 16
_NIDX = _BPW // 128


_UR = 4


@functools.partial(
    pl.kernel,
    mesh=plsc.VectorSubcoreMesh(core_axis_name="c", subcore_axis_name="s"),
    out_type=jax.ShapeDtypeStruct((_NW, 16), jnp.float32),
    scratch_types=[
        pltpu.VMEM((2 * _CH,), jnp.float32),
        pltpu.VMEM((_SR,), jnp.float32),
        pltpu.VMEM((_BPW,), jnp.int32),
        pltpu.VMEM((_NIDX, 128), jnp.int32),
        pltpu.VMEM((_NIDX, 128), jnp.int32),
        pltpu.VMEM((_NIDX, 128), jnp.float32),
        pltpu.VMEM((_NIDX, 128), jnp.float32),
        pltpu.VMEM((16,), jnp.float32),
        pltpu.SemaphoreType.DMA,
        pltpu.SemaphoreType.DMA,
    ],
)
def _loss_kernel(xflat_hbm, tgt_hbm, w_hbm, w2_hbm, out_hbm,
                 xb_v, w2_v, tgt_v, idx_v, tdx_v, xs_v, ws_v, o_v,
                 gsem, ssem):
    wid = lax.axis_index("s") * _NC + lax.axis_index("c")
    base = wid * _FPW

    pltpu.sync_copy(tgt_hbm.at[pl.ds(wid * _BPW, _BPW)], tgt_v)
    for i in range(_NCHUNK):
        t16 = tgt_v[pl.ds(i * 16, 16)]
        rows = wid * _BPW + i * 16 + lax.broadcasted_iota(jnp.int32, (16,), 0)
        idx_v[i // 8, pl.ds((i % 8) * 16, 16)] = rows * _D + t16
        tdx_v[i // 8, pl.ds((i % 8) * 16, 16)] = t16
    gcopies = [
        pltpu.async_copy(xflat_hbm.at[idx_v.at[j]], xs_v.at[j], gsem)
        for j in range(_NIDX)
    ] + [
        pltpu.async_copy(w_hbm.at[tdx_v.at[j]], ws_v.at[j], gsem)
        for j in range(_NIDX)
    ]

    pltpu.sync_copy(w2_hbm, w2_v)
    pltpu.async_copy(
        xflat_hbm.at[pl.ds(base, _CH)], xb_v.at[pl.ds(0, _CH)], ssem)

    def _chunk_body(k, accs):
        half = lax.rem(k, 2)

        @pl.when(k + 1 < _NCHK)
        def _start_next():
            pltpu.async_copy(
                xflat_hbm.at[pl.ds(base + (k + 1) * _CH, _CH)],
                xb_v.at[pl.ds(lax.rem(k + 1, 2) * _CH, _CH)], ssem)

        pltpu.make_async_copy(
            xflat_hbm.at[pl.ds(0, _CH)], xb_v.at[pl.ds(0, _CH)], ssem).wait()

        def _group_body(g, accs):
            accs = list(accs)
            off0 = half * _CH + g * (_UR * _SR)
            n = 0
            for c2 in range(_SR // 16):
                wv = w2_v[pl.ds(c2 * 16, 16)]
                for u in range(_UR):
                    xv = xb_v[pl.ds(off0 + u * _SR + c2 * 16, 16)]
                    accs[n % _NACC] = accs[n % _NACC] + wv * (xv * xv)
                    n += 1
            return tuple(accs)

        return lax.fori_loop(0, _CH // (_UR * _SR), _group_body, accs)

    accs = lax.fori_loop(
        0, _NCHK, _chunk_body,
        tuple(jnp.zeros((16,), jnp.float32) for _ in range(_NACC)))

    for cp in gcopies:
        cp.wait()
    acc = accs[0]
    for a in accs[1:]:
        acc = acc + a
    for i in range(_NCHUNK):
        x16 = xs_v[i // 8, pl.ds((i % 8) * 16, 16)]
        w16 = ws_v[i // 8, pl.ds((i % 8) * 16, 16)]
        acc = acc + w16 * (1.0 - 2.0 * x16)
    o_v[...] = acc
    pltpu.sync_copy(o_v, out_hbm.at[wid])


def kernel(inputs, targets, loss_weights):
    scale = (targets[0] * 0 + 1).astype(jnp.float32)
    xflat = (inputs * scale).reshape(_B * _D)
    w2 = jnp.concatenate([loss_weights, loss_weights])
    parts = _loss_kernel(xflat, targets, loss_weights, w2)
    return jnp.sum(parts) / jnp.float32(_B * _D)
